# baseline (speedup 1.0000x reference)
import functools

import jax
import jax.numpy as jnp
from jax import lax
from jax.experimental import pallas as pl
from jax.experimental.pallas import tpu as pltpu

B, L, A = 1024, 50, 100000
STATE_DIM, ACTION_DIM, H = 128, 64, 128
AT = 2048
import numpy as np
NEG_INF = np.float32(-np.inf)


def _heur_body(avg_ref, img_ref, ae_ref, taken_ref, heur_ref):
    avg = avg_ref[...]
    img = img_ref[...]
    ae = ae_ref[...]
    h1 = lax.dot_general(avg, ae, (((1,), (1,)), ((), ())),
                         preferred_element_type=jnp.float32)
    h2 = lax.dot_general(img, ae, (((1,), (1,)), ((), ())),
                         preferred_element_type=jnp.float32)
    heur = h1 * 0.5 + h2 * 0.5
    heur_ref[...] = jnp.where(taken_ref[...] == 1.0, NEG_INF, heur)


def _out_body(latent_ref, ae_ref, heur_ref, mask_ref, out_ref):
    latent = latent_ref[...]
    ae = ae_ref[...]
    pref = lax.dot_general(latent, ae, (((1,), (1,)), ((), ())),
                           preferred_element_type=jnp.float32)
    heur = heur_ref[...]
    out_ref[...] = jnp.where(mask_ref[...] != 0, pref + heur, NEG_INF)


def _dense_chain(state, action_seq, seq_masks, p):
    half = state.shape[-1] // 2
    img = state[..., :half]
    i = jax.nn.relu(img @ p['ci_W1'].T + p['ci_b1'])
    i = i @ p['ci_W2'].T + p['ci_b2']
    h = action_seq
    for l in range(3):
        Wih, Whh = p[f'gru_Wih{l}'], p[f'gru_Whh{l}']
        bih, bhh = p[f'gru_bih{l}'], p[f'gru_bhh{l}']

        def step(hc, x_t, Wih=Wih, Whh=Whh, bih=bih, bhh=bhh):
            gi = x_t @ Wih.T + bih
            gh = hc @ Whh.T + bhh
            i_r, i_z, i_n = jnp.split(gi, 3, axis=-1)
            h_r, h_z, h_n = jnp.split(gh, 3, axis=-1)
            r = jax.nn.sigmoid(i_r + h_r)
            z = jax.nn.sigmoid(i_z + h_z)
            n = jnp.tanh(i_n + r * h_n)
            hn = (1.0 - z) * n + z * hc
            return hn, hn

        _, hs = lax.scan(step, i, jnp.swapaxes(h, 0, 1))
        h = jnp.swapaxes(hs, 0, 1)
    idx = jnp.argmax(jnp.cumsum(seq_masks, axis=1), axis=1)
    logits = h[jnp.arange(h.shape[0]), idx]
    t = logits @ p['tp_W'].T + p['tp_b']
    x = t @ p['h_W1'].T + p['h_b1']
    mu = jnp.mean(x, axis=0)
    var = jnp.var(x, axis=0)
    x = (x - mu) / jnp.sqrt(var + 1e-5) * p['bn_g'] + p['bn_b']
    x = jax.nn.relu(x)
    return x @ p['h_W2'].T + p['h_b2']


def kernel(state, actions_encode, taken_actions, action_seq, seq_masks, params):
    p = params
    half = state.shape[-1] // 2
    img = state[..., :half]
    latent = _dense_chain(state, action_seq, seq_masks, p)
    avg = jnp.sum(action_seq, axis=1) / jnp.sum(seq_masks, axis=1, keepdims=True)

    grid = ((A + AT - 1) // AT,)
    heur = pl.pallas_call(
        _heur_body,
        grid=grid,
        in_specs=[
            pl.BlockSpec((B, ACTION_DIM), lambda a: (0, 0)),
            pl.BlockSpec((B, ACTION_DIM), lambda a: (0, 0)),
            pl.BlockSpec((AT, ACTION_DIM), lambda a: (a, 0)),
            pl.BlockSpec((B, AT), lambda a: (0, a)),
        ],
        out_specs=pl.BlockSpec((B, AT), lambda a: (0, a)),
        out_shape=jax.ShapeDtypeStruct((B, A), jnp.float32),
    )(avg, img, actions_encode, taken_actions)

    _, top_idx = lax.top_k(heur, 50)
    mask = jnp.zeros((B, A), jnp.uint8).at[
        jnp.arange(B)[:, None], top_idx].set(1, mode='drop')

    out = pl.pallas_call(
        _out_body,
        grid=grid,
        in_specs=[
            pl.BlockSpec((B, ACTION_DIM), lambda a: (0, 0)),
            pl.BlockSpec((AT, ACTION_DIM), lambda a: (a, 0)),
            pl.BlockSpec((B, AT), lambda a: (0, a)),
            pl.BlockSpec((B, AT), lambda a: (0, a)),
        ],
        out_specs=pl.BlockSpec((B, AT), lambda a: (0, a)),
        out_shape=jax.ShapeDtypeStruct((B, A), jnp.float32),
    )(latent, actions_encode, heur, mask)
    return out
